# Initial kernel scaffold; baseline (speedup 1.0000x reference)
import jax
import jax.numpy as jnp
from jax.experimental import pallas as pl


def kernel(tokens, table):
    raise NotImplementedError("write your pallas kernel here")


# speedup vs baseline: 1.5037x; 1.5037x over previous
import functools

import jax
import jax.numpy as jnp
from jax import lax
from jax.experimental import pallas as pl
from jax.experimental.pallas import tpu as pltpu
from jax.experimental.pallas import tpu_sc as plsc

VOCAB = 6561
D = 1024
B = 1024 * 50
NC, NS = 2, 16
NW = NC * NS
BPW = B // NW
C = 40
NCHUNK = BPW // C


def _emb_body(tokens_hbm, table_hbm, out_hbm, idx_v, buf, gsem, osem):
    wid = lax.axis_index("s") * NC + lax.axis_index("c")
    base = pl.multiple_of(wid * BPW, 8)
    pltpu.sync_copy(tokens_hbm.at[pl.ds(base, BPW)], idx_v)

    @pl.loop(0, NCHUNK)
    def _chunk(g):
        off = pl.multiple_of(g * C, 8)
        pltpu.async_copy(table_hbm.at[idx_v.at[pl.ds(off, C)]], buf, gsem).wait()
        pltpu.async_copy(buf, out_hbm.at[pl.ds(base + off, C)], osem).wait()


@jax.jit
def _emb(tokens_flat, table):
    run = pl.kernel(
        _emb_body,
        out_type=jax.ShapeDtypeStruct((B, D), jnp.float32),
        mesh=plsc.VectorSubcoreMesh(core_axis_name="c", subcore_axis_name="s"),
        scratch_types=[
            pltpu.VMEM((BPW,), jnp.int32),
            pltpu.VMEM((C, D), jnp.float32),
            pltpu.SemaphoreType.DMA,
            pltpu.SemaphoreType.DMA,
        ],
    )
    return run(tokens_flat, table)


def kernel(tokens, table):
    bt = tokens.shape
    out = _emb(tokens.reshape(-1).astype(jnp.int32), table)
    return out.reshape(*bt, D)


# speedup vs baseline: 1.5729x; 1.0460x over previous
import functools

import jax
import jax.numpy as jnp
from jax import lax
from jax.experimental import pallas as pl
from jax.experimental.pallas import tpu as pltpu
from jax.experimental.pallas import tpu_sc as plsc

VOCAB = 6561
D = 1024
B = 1024 * 50
NC, NS = 2, 16
NW = NC * NS
BPW = B // NW
C = 40
NCHUNK = BPW // C


def _emb_body(tokens_hbm, table_hbm, out_hbm, idx_v, buf0, buf1, gs0, gs1, os0, os1):
    wid = lax.axis_index("s") * NC + lax.axis_index("c")
    base = pl.multiple_of(wid * BPW, 8)
    pltpu.sync_copy(tokens_hbm.at[pl.ds(base, BPW)], idx_v)

    bufs = (buf0, buf1)
    gsems = (gs0, gs1)
    osems = (os0, os1)

    def start_gather(i, b):
        off = pl.multiple_of(i * C, 8)
        pltpu.async_copy(table_hbm.at[idx_v.at[pl.ds(off, C)]], bufs[b], gsems[b])

    def wait_gather(b):
        pltpu.make_async_copy(table_hbm.at[pl.ds(0, C)], bufs[b], gsems[b]).wait()

    def start_scatter(i, b):
        off = pl.multiple_of(base + i * C, 8)
        pltpu.async_copy(bufs[b], out_hbm.at[pl.ds(off, C)], osems[b])

    def wait_scatter(b):
        pltpu.make_async_copy(bufs[b], out_hbm.at[pl.ds(base, C)], osems[b]).wait()

    start_gather(0, 0)

    @pl.loop(0, NCHUNK // 2)
    def _pair(s):
        i0 = s * 2

        @pl.when(s > 0)
        def _():
            wait_scatter(1)
        start_gather(i0 + 1, 1)
        wait_gather(0)
        start_scatter(i0, 0)

        @pl.when(s < NCHUNK // 2 - 1)
        def _():
            wait_scatter(0)
            start_gather(i0 + 2, 0)
        wait_gather(1)
        start_scatter(i0 + 1, 1)

    wait_scatter(0)
    wait_scatter(1)


@jax.jit
def _emb(tokens_flat, table):
    run = pl.kernel(
        _emb_body,
        out_type=jax.ShapeDtypeStruct((B, D), jnp.float32),
        mesh=plsc.VectorSubcoreMesh(core_axis_name="c", subcore_axis_name="s"),
        scratch_types=[
            pltpu.VMEM((BPW,), jnp.int32),
            pltpu.VMEM((C, D), jnp.float32),
            pltpu.VMEM((C, D), jnp.float32),
            pltpu.SemaphoreType.DMA,
            pltpu.SemaphoreType.DMA,
            pltpu.SemaphoreType.DMA,
            pltpu.SemaphoreType.DMA,
        ],
    )
    return run(tokens_flat, table)


def kernel(tokens, table):
    bt = tokens.shape
    out = _emb(tokens.reshape(-1).astype(jnp.int32), table)
    return out.reshape(*bt, D)
